# baseline (speedup 1.0000x reference)
import functools

import jax
import jax.numpy as jnp
from jax import lax
from jax.experimental import pallas as pl
from jax.experimental.pallas import tpu as pltpu
from jax.experimental.pallas import tpu_sc as plsc

ROWS = 4096
COLS = 200
D = 32
GRP = 128 // D
NC = 2
NS = 16
NW = NC * NS
RPW = ROWS // NW
L = 16
OFFS = tuple(range(0, COLS - L, L)) + (COLS - L,)

_mesh = plsc.VectorSubcoreMesh(core_axis_name="c", subcore_axis_name="s")


@functools.partial(
    pl.kernel,
    mesh=_mesh,
    out_type=jax.ShapeDtypeStruct((ROWS, COLS, D), jnp.float32),
    scratch_types=[
        [pltpu.VMEM((COLS,), jnp.int32)] * 2,
        [pltpu.VMEM((COLS,), jnp.int32)] * 2,
        [pltpu.VMEM((COLS, 128), jnp.float32)] * 2,
        [pltpu.VMEM((COLS, D), jnp.float32)] * 2,
        [pltpu.SemaphoreType.DMA] * 2,
        [pltpu.SemaphoreType.DMA] * 2,
    ],
    compiler_params=pltpu.CompilerParams(
        use_tc_tiling_on_sc=True, needs_layout_passes=False
    ),
)
def _gather(idx_hbm, table_hbm, out_hbm, ids_v, ridx, gbuf, obuf, gsem, wsem):
    wid = lax.axis_index("s") * NC + lax.axis_index("c")
    base = wid * RPW

    def compute_ridx(j, p):
        pltpu.sync_copy(idx_hbm.at[base + j], ids_v[p])
        for o in OFFS:
            ids = ids_v[p][pl.ds(o, L)]
            ridx[p][pl.ds(o, L)] = lax.shift_right_logical(ids, 2)

    def gcopy(j, p):
        return pltpu.make_async_copy(table_hbm.at[ridx[p]], gbuf[p], gsem[p])

    def extract(j, p):
        lane = lax.iota(jnp.int32, L)
        for o in OFFS:
            ids = ids_v[p][pl.ds(o, L)]
            colb = lax.bitwise_and(ids, GRP - 1) * D
            jv = o + lane
            for c in range(D):
                vals = plsc.load_gather(gbuf[p], [jv, colb + c])
                plsc.store_scatter(obuf[p], [jv, jnp.full((L,), c, jnp.int32)], vals)

    def wcopy(j, p):
        return pltpu.make_async_copy(obuf[p], out_hbm.at[base + j], wsem[p])

    for p in range(2):
        compute_ridx(p, p)
        gcopy(p, p).start()

    def turn(t, carry):
        for p in range(2):
            j = 2 * t + p
            gcopy(j, p).wait()
            extract(j, p)
            wcopy(j, p).start()
            wcopy(j, p).wait()

            @pl.when(j + 2 < RPW)
            def _():
                compute_ridx(j + 2, p)
                gcopy(j + 2, p).start()

        return carry

    lax.fori_loop(0, RPW // 2, turn, 0)


def kernel(token_ids, weights):
    w128 = weights.reshape(weights.shape[0] // GRP, 128)
    return _gather(token_ids, w128)


# speedup vs baseline: 1.1829x; 1.1829x over previous
import functools

import jax
import jax.numpy as jnp
from jax import lax
from jax.experimental import pallas as pl
from jax.experimental.pallas import tpu as pltpu
from jax.experimental.pallas import tpu_sc as plsc

ROWS = 4096
COLS = 200
D = 32
GRP = 128 // D
NC = 2
NS = 16
NW = NC * NS
RPW = ROWS // NW
L = 16
OFFS = tuple(range(0, COLS - L, L)) + (COLS - L,)

_mesh = plsc.VectorSubcoreMesh(core_axis_name="c", subcore_axis_name="s")


@functools.partial(
    pl.kernel,
    mesh=_mesh,
    out_type=jax.ShapeDtypeStruct((ROWS, COLS, D), jnp.float32),
    scratch_types=[
        [pltpu.VMEM((COLS,), jnp.int32)] * 2,
        [pltpu.VMEM((COLS,), jnp.int32)] * 2,
        [pltpu.VMEM((COLS, 128), jnp.float32)] * 2,
        [pltpu.VMEM((COLS, D), jnp.float32)] * 2,
        [pltpu.SemaphoreType.DMA] * 2,
        [pltpu.SemaphoreType.DMA] * 2,
    ],
    compiler_params=pltpu.CompilerParams(
        use_tc_tiling_on_sc=True, needs_layout_passes=False
    ),
)
def _gather(idx_hbm, table_hbm, out_hbm, ids_v, ridx, gbuf, obuf, gsem, wsem):
    wid = lax.axis_index("s") * NC + lax.axis_index("c")
    base = wid * RPW

    def compute_ridx(j, p):
        pltpu.sync_copy(idx_hbm.at[base + j], ids_v[p])
        for o in OFFS:
            ids = ids_v[p][pl.ds(o, L)]
            ridx[p][pl.ds(o, L)] = lax.shift_right_logical(ids, 2)

    def gcopy(j, p):
        return pltpu.make_async_copy(table_hbm.at[ridx[p]], gbuf[p], gsem[p])

    def extract(j, p):
        lane = lax.iota(jnp.int32, L)
        for o in OFFS:
            ids = ids_v[p][pl.ds(o, L)]
            colb = lax.bitwise_and(ids, GRP - 1) * D
            jv = o + lane
            for cb in range(0, D, 8):
                vals = [
                    plsc.load_gather(gbuf[p], [jv, colb + (cb + k)])
                    for k in range(8)
                ]
                for k in range(8):
                    plsc.store_scatter(
                        obuf[p],
                        [jv, jnp.full((L,), cb + k, jnp.int32)],
                        vals[k],
                    )

    def wcopy(j, p):
        return pltpu.make_async_copy(obuf[p], out_hbm.at[base + j], wsem[p])

    for p in range(2):
        compute_ridx(p, p)
        gcopy(p, p).start()

    def turn(t, carry):
        for p in range(2):
            j = 2 * t + p
            gcopy(j, p).wait()
            extract(j, p)
            wcopy(j, p).start()
            wcopy(j, p).wait()

            @pl.when(j + 2 < RPW)
            def _():
                compute_ridx(j + 2, p)
                gcopy(j + 2, p).start()

        return carry

    lax.fori_loop(0, RPW // 2, turn, 0)


def kernel(token_ids, weights):
    w128 = weights.reshape(weights.shape[0] // GRP, 128)
    return _gather(token_ids, w128)


# speedup vs baseline: 1.9861x; 1.6790x over previous
import functools

import jax
import jax.numpy as jnp
from jax import lax
from jax.experimental import pallas as pl
from jax.experimental.pallas import tpu as pltpu
from jax.experimental.pallas import tpu_sc as plsc

ROWS = 4096
COLS = 200
D = 32
NC = 2
NS = 16
NW = NC * NS
RPW = ROWS // NW
NSLOT = 4

_mesh = plsc.VectorSubcoreMesh(core_axis_name="c", subcore_axis_name="s")


@functools.partial(
    pl.kernel,
    mesh=_mesh,
    out_type=jax.ShapeDtypeStruct((ROWS, COLS, D), jnp.float32),
    scratch_types=[
        pltpu.VMEM((RPW, COLS), jnp.int32),
        [pltpu.VMEM((COLS, D), jnp.float32)] * NSLOT,
        [pltpu.SemaphoreType.DMA] * NSLOT,
        [pltpu.SemaphoreType.DMA] * NSLOT,
    ],
    compiler_params=pltpu.CompilerParams(
        use_tc_tiling_on_sc=False, skip_device_barrier=True
    ),
)
def _gather(idx_hbm, table_hbm, out_hbm, idx_v, bufs, gsem, wsem):
    wid = lax.axis_index("s") * NC + lax.axis_index("c")
    base = wid * RPW

    pltpu.sync_copy(idx_hbm.at[pl.ds(base, RPW)], idx_v)

    def gcopy(j, p):
        return pltpu.make_async_copy(table_hbm.at[idx_v.at[j]], bufs[p], gsem[p])

    def wcopy(j, p):
        return pltpu.make_async_copy(bufs[p], out_hbm.at[base + j], wsem[p])

    for p in range(NSLOT):
        gcopy(p, p).start()

    def turn(t, carry):
        for p in range(NSLOT):
            j = NSLOT * t + p
            gcopy(j, p).wait()
            wcopy(j, p).start()
            wcopy(j, p).wait()

            @pl.when(j + NSLOT < RPW)
            def _():
                gcopy(j + NSLOT, p).start()

        return carry

    lax.fori_loop(0, RPW // NSLOT, turn, 0)


def kernel(token_ids, weights):
    return _gather(token_ids, weights)


# speedup vs baseline: 1.9867x; 1.0003x over previous
import functools

import jax
import jax.numpy as jnp
from jax import lax
from jax.experimental import pallas as pl
from jax.experimental.pallas import tpu as pltpu
from jax.experimental.pallas import tpu_sc as plsc

ROWS = 4096
COLS = 200
D = 32
NC = 2
NS = 16
NW = NC * NS
RPW = ROWS // NW
NSLOT = 4

_mesh = plsc.VectorSubcoreMesh(core_axis_name="c", subcore_axis_name="s")


@functools.partial(
    pl.kernel,
    mesh=_mesh,
    out_type=jax.ShapeDtypeStruct((ROWS, COLS, D), jnp.float32),
    scratch_types=[
        pltpu.VMEM((RPW, COLS), jnp.int32),
        [pltpu.VMEM((COLS, D), jnp.float32)] * NSLOT,
        [pltpu.SemaphoreType.DMA] * NSLOT,
        [pltpu.SemaphoreType.DMA] * NSLOT,
    ],
    compiler_params=pltpu.CompilerParams(use_tc_tiling_on_sc=False),
)
def _gather(idx_hbm, table_hbm, out_hbm, idx_v, bufs, gsem, wsem):
    wid = lax.axis_index("s") * NC + lax.axis_index("c")
    base = wid * RPW

    pltpu.sync_copy(idx_hbm.at[pl.ds(base, RPW)], idx_v)

    def gcopy(j, p):
        return pltpu.make_async_copy(table_hbm.at[idx_v.at[j]], bufs[p], gsem[p])

    def wcopy(j, p):
        return pltpu.make_async_copy(bufs[p], out_hbm.at[base + j], wsem[p])

    for p in range(NSLOT):
        gcopy(p, p).start()

    def turn(t, carry):
        for p in range(NSLOT):
            j = NSLOT * t + p
            gcopy(j, p).wait()
            wcopy(j, p).start()
            wcopy(j, p).wait()

            @pl.when(j + NSLOT < RPW)
            def _():
                gcopy(j + NSLOT, p).start()

        return carry

    lax.fori_loop(0, RPW // NSLOT, turn, 0)


def kernel(token_ids, weights):
    return _gather(token_ids, weights)
